# baseline (speedup 1.0000x reference)
import jax
import jax.numpy as jnp
from jax import lax
from jax.experimental import pallas as pl
from jax.experimental.pallas import tpu as pltpu
from jax.experimental.pallas import tpu_sc as plsc

_NF = 26
_NP = 325
_W = 1024 * 64
_NC = 2
_NS = 16
_NW = _NC * _NS
_C = _W // _NW
_L = 16


def _sc_body(x_hbm, out_hbm, xv, ob, sem):
    wid = lax.axis_index("s") * _NC + lax.axis_index("c")
    base = wid * _C
    pltpu.sync_copy(x_hbm.at[:, pl.ds(base, _C)], xv)

    def pair_step(p, carry):
        i, j = carry
        slot = lax.rem(p, 2)

        @pl.when(p >= 2)
        def _wait_prev():
            pltpu.make_async_copy(
                ob.at[slot], out_hbm.at[pl.ds(p - 2, 1), pl.ds(base, _C)],
                sem.at[slot],
            ).wait()

        def vec_step(v, acc):
            sl = pl.ds(v * _L, _L)
            ob[slot, 0, sl] = xv[i, sl] * xv[j, sl]
            return acc

        lax.fori_loop(0, _C // _L, vec_step, 0, unroll=8)

        pltpu.async_copy(
            ob.at[slot], out_hbm.at[pl.ds(p, 1), pl.ds(base, _C)], sem.at[slot]
        )

        jn = j + 1
        wrap = jn == _NF
        inx = jnp.where(wrap, i + 1, i)
        jn = jnp.where(wrap, inx + 1, jn)
        return inx, jn

    lax.fori_loop(0, _NP, pair_step, (jnp.int32(0), jnp.int32(1)))

    pltpu.make_async_copy(
        ob.at[1], out_hbm.at[pl.ds(_NP - 2, 1), pl.ds(base, _C)], sem.at[1]
    ).wait()
    pltpu.make_async_copy(
        ob.at[0], out_hbm.at[pl.ds(_NP - 1, 1), pl.ds(base, _C)], sem.at[0]
    ).wait()


def kernel(x):
    xf = x.reshape(_NF, _W)
    k = pl.kernel(
        _sc_body,
        out_type=jax.ShapeDtypeStruct((_NP, _W), jnp.float32),
        mesh=plsc.VectorSubcoreMesh(core_axis_name="c", subcore_axis_name="s"),
        scratch_types=[
            pltpu.VMEM((_NF, _C), jnp.float32),
            pltpu.VMEM((2, 1, _C), jnp.float32),
            pltpu.SemaphoreType.DMA((2,)),
        ],
    )
    out = k(xf)
    return out.reshape(_NP, 1024, 64)


# speedup vs baseline: 1.5904x; 1.5904x over previous
import jax
import jax.numpy as jnp
from jax import lax
from jax.experimental import pallas as pl
from jax.experimental.pallas import tpu as pltpu
from jax.experimental.pallas import tpu_sc as plsc

_NF = 26
_NP = 325
_W = 1024 * 64
_NC = 2
_NS = 16
_NW = _NC * _NS
_C = _W // _NW
_L = 16
_G = 8
_NB = _NP // _G

_PAIRS = [(i, j) for i in range(_NF) for j in range(i + 1, _NF)]


def _sc_body(x_hbm, out_hbm, xv, ob0, ob1, sem):
    wid = lax.axis_index("s") * _NC + lax.axis_index("c")
    base = wid * _C
    pltpu.sync_copy(x_hbm.at[:, pl.ds(base, _C)], xv)

    bufs = (ob0, ob1)

    def compute_block(buf, block_pairs):
        def vec_step(v, acc):
            sl = pl.ds(v * _L, _L)
            for g, (i, j) in enumerate(block_pairs):
                buf[g, sl] = xv[i, sl] * xv[j, sl]
            return acc

        lax.fori_loop(0, _C // _L, vec_step, 0)

    for b in range(_NB):
        buf = bufs[b % 2]
        p0 = b * _G
        if b >= 2:
            pltpu.make_async_copy(
                buf,
                out_hbm.at[pl.ds((b - 2) * _G, _G), pl.ds(base, _C)],
                sem.at[b % 2],
            ).wait()
        compute_block(buf, _PAIRS[p0:p0 + _G])
        pltpu.async_copy(
            buf,
            out_hbm.at[pl.ds(p0, _G), pl.ds(base, _C)],
            sem.at[b % 2],
        )

    for b in (_NB - 2, _NB - 1):
        pltpu.make_async_copy(
            bufs[b % 2],
            out_hbm.at[pl.ds(b * _G, _G), pl.ds(base, _C)],
            sem.at[b % 2],
        ).wait()

    tail0 = _NB * _G
    ntail = _NP - tail0
    compute_block(ob0, _PAIRS[tail0:])
    for g in range(ntail):
        pltpu.async_copy(
            ob0.at[pl.ds(g, 1)],
            out_hbm.at[pl.ds(tail0 + g, 1), pl.ds(base, _C)],
            sem.at[0],
        )
    for g in range(ntail):
        pltpu.make_async_copy(
            ob0.at[pl.ds(g, 1)],
            out_hbm.at[pl.ds(tail0 + g, 1), pl.ds(base, _C)],
            sem.at[0],
        ).wait()


def kernel(x):
    xf = x.reshape(_NF, _W)
    k = pl.kernel(
        _sc_body,
        out_type=jax.ShapeDtypeStruct((_NP, _W), jnp.float32),
        mesh=plsc.VectorSubcoreMesh(core_axis_name="c", subcore_axis_name="s"),
        scratch_types=[
            pltpu.VMEM((_NF, _C), jnp.float32),
            pltpu.VMEM((_G, _C), jnp.float32),
            pltpu.VMEM((_G, _C), jnp.float32),
            pltpu.SemaphoreType.DMA((2,)),
        ],
    )
    out = k(xf)
    return out.reshape(_NP, 1024, 64)


# speedup vs baseline: 1.9646x; 1.2353x over previous
import jax
import jax.numpy as jnp
from jax import lax
from jax.experimental import pallas as pl
from jax.experimental.pallas import tpu as pltpu
from jax.experimental.pallas import tpu_sc as plsc

_NF = 26
_NP = 325
_B = 1024
_D = 64
_NC = 2
_NS = 16
_NW = _NC * _NS
_R = 16
_L = 16
_G = 13
_NB = _NP // _G

_PAIRS = [(i, j) for i in range(_NF) for j in range(i + 1, _NF)]


def _sc_body(x_hbm, out_hbm, xv, ob0, ob1, sem):
    wid = lax.axis_index("s") * _NC + lax.axis_index("c")
    bufs = (ob0, ob1)

    def compute_block(buf, block_pairs):
        def row_step(r, acc):
            for g, (i, j) in enumerate(block_pairs):
                for c in range(_D // _L):
                    sl = pl.ds(c * _L, _L)
                    buf[g, r, sl] = xv[i, r, sl] * xv[j, r, sl]
            return acc

        lax.fori_loop(0, _R, row_step, 0)

    def pass_step(ps, acc):
        r0 = pl.multiple_of((ps * _NW + wid) * _R, _R)
        r0_prev = pl.multiple_of((ps * _NW + wid - _NW) * _R, _R)

        pltpu.sync_copy(x_hbm.at[:, pl.ds(r0, _R), :], xv)

        for b in range(_NB):
            buf = bufs[b % 2]
            p0 = b * _G
            if b >= 2:
                pltpu.make_async_copy(
                    buf,
                    out_hbm.at[pl.ds((b - 2) * _G, _G), pl.ds(r0, _R), :],
                    sem.at[b % 2],
                ).wait()
            else:
                @pl.when(ps >= 1)
                def _wait_prev_pass(buf=buf, b=b):
                    pltpu.make_async_copy(
                        buf,
                        out_hbm.at[
                            pl.ds((_NB - 2 + b) * _G, _G),
                            pl.ds(r0_prev, _R), :,
                        ],
                        sem.at[b % 2],
                    ).wait()
            compute_block(buf, _PAIRS[p0:p0 + _G])
            pltpu.async_copy(
                buf,
                out_hbm.at[pl.ds(p0, _G), pl.ds(r0, _R), :],
                sem.at[b % 2],
            )
        return acc

    lax.fori_loop(0, 2, pass_step, 0)

    r0_last = (_NW + wid) * _R
    for b in (_NB - 2, _NB - 1):
        pltpu.make_async_copy(
            bufs[b % 2],
            out_hbm.at[pl.ds(b * _G, _G), pl.ds(r0_last, _R), :],
            sem.at[b % 2],
        ).wait()


def kernel(x):
    k = pl.kernel(
        _sc_body,
        out_type=jax.ShapeDtypeStruct((_NP, _B, _D), jnp.float32),
        mesh=plsc.VectorSubcoreMesh(core_axis_name="c", subcore_axis_name="s"),
        scratch_types=[
            pltpu.VMEM((_NF, _R, _D), jnp.float32),
            pltpu.VMEM((_G, _R, _D), jnp.float32),
            pltpu.VMEM((_G, _R, _D), jnp.float32),
            pltpu.SemaphoreType.DMA((2,)),
        ],
    )
    return k(x)
